# baseline (speedup 1.0000x reference)
import jax
import jax.numpy as jnp
from jax.experimental import pallas as pl

HIDDEN = 2048
NUM_EXPERTS = 64
TOPK = 8
TOPK_SCALING_FACTOR = 2.5
BLK = 256


def _router_body(x_ref, wt_ref, b_ref, probs_ref, map_ref):
    x = x_ref[...]
    logits = jax.lax.dot_general(
        x, wt_ref[...], (((1,), (1,)), ((), ())),
        preferred_element_type=jnp.float32,
    )
    sp = jnp.maximum(logits, 0.0) + jnp.log(1.0 + jnp.exp(-jnp.abs(logits)))
    scores = jnp.sqrt(sp)
    sel = scores + b_ref[...]
    iota = jax.lax.broadcasted_iota(jnp.int32, sel.shape, 1)
    mask = jnp.zeros(sel.shape, jnp.bool_)
    work = sel
    for _ in range(TOPK):
        idx = jnp.argmax(work, axis=1)[:, None]
        chosen = iota == idx
        mask = jnp.logical_or(mask, chosen)
        work = jnp.where(chosen, -jnp.inf, work)
    w = jnp.where(mask, scores, 0.0)
    denom = jnp.clip(jnp.sum(w, axis=1, keepdims=True), 1e-12, None)
    probs_ref[...] = jnp.where(mask, scores * (TOPK_SCALING_FACTOR / denom), 0.0)
    map_ref[...] = mask


def kernel(hidden, weight, expert_bias):
    flat = hidden.reshape(-1, HIDDEN)
    n = flat.shape[0]
    bias = expert_bias.reshape(1, NUM_EXPERTS)
    probs, rmap = pl.pallas_call(
        _router_body,
        grid=(n // BLK,),
        in_specs=[
            pl.BlockSpec((BLK, HIDDEN), lambda i: (i, 0)),
            pl.BlockSpec((NUM_EXPERTS, HIDDEN), lambda i: (0, 0)),
            pl.BlockSpec((1, NUM_EXPERTS), lambda i: (0, 0)),
        ],
        out_specs=[
            pl.BlockSpec((BLK, NUM_EXPERTS), lambda i: (i, 0)),
            pl.BlockSpec((BLK, NUM_EXPERTS), lambda i: (i, 0)),
        ],
        out_shape=[
            jax.ShapeDtypeStruct((n, NUM_EXPERTS), jnp.float32),
            jax.ShapeDtypeStruct((n, NUM_EXPERTS), jnp.bool_),
        ],
    )(flat, weight, bias)
    return probs, rmap


# speedup vs baseline: 1.5591x; 1.5591x over previous
import jax
import jax.numpy as jnp
from jax.experimental import pallas as pl

HIDDEN = 2048
NUM_EXPERTS = 64
TOPK = 8
TOPK_SCALING_FACTOR = 2.5
BLK = 1024


def _router_body(x_ref, wt_ref, b_ref, probs_ref, map_ref):
    x = x_ref[...]
    logits = jax.lax.dot_general(
        x, wt_ref[...], (((1,), (1,)), ((), ())),
        preferred_element_type=jnp.float32,
    )
    sp = jnp.maximum(logits, 0.0) + jnp.log(1.0 + jnp.exp(-jnp.abs(logits)))
    scores = jnp.sqrt(sp)
    sel = scores + b_ref[...]
    iota = jax.lax.broadcasted_iota(jnp.int32, sel.shape, 1)
    mask = jnp.zeros(sel.shape, jnp.bool_)
    work = sel
    for _ in range(TOPK):
        idx = jnp.argmax(work, axis=1)[:, None]
        chosen = iota == idx
        mask = jnp.logical_or(mask, chosen)
        work = jnp.where(chosen, -jnp.inf, work)
    w = jnp.where(mask, scores, 0.0)
    denom = jnp.clip(jnp.sum(w, axis=1, keepdims=True), 1e-12, None)
    probs_ref[...] = jnp.where(mask, scores * (TOPK_SCALING_FACTOR / denom), 0.0)
    map_ref[...] = mask


def kernel(hidden, weight, expert_bias):
    flat = hidden.reshape(-1, HIDDEN)
    n = flat.shape[0]
    bias = expert_bias.reshape(1, NUM_EXPERTS)
    probs, rmap = pl.pallas_call(
        _router_body,
        grid=(n // BLK,),
        in_specs=[
            pl.BlockSpec((BLK, HIDDEN), lambda i: (i, 0)),
            pl.BlockSpec((NUM_EXPERTS, HIDDEN), lambda i: (0, 0)),
            pl.BlockSpec((1, NUM_EXPERTS), lambda i: (0, 0)),
        ],
        out_specs=[
            pl.BlockSpec((BLK, NUM_EXPERTS), lambda i: (i, 0)),
            pl.BlockSpec((BLK, NUM_EXPERTS), lambda i: (i, 0)),
        ],
        out_shape=[
            jax.ShapeDtypeStruct((n, NUM_EXPERTS), jnp.float32),
            jax.ShapeDtypeStruct((n, NUM_EXPERTS), jnp.bool_),
        ],
    )(flat, weight, bias)
    return probs, rmap


# speedup vs baseline: 1.6255x; 1.0426x over previous
import jax
import jax.numpy as jnp
from jax.experimental import pallas as pl

HIDDEN = 2048
NUM_EXPERTS = 64
TOPK = 8
TOPK_SCALING_FACTOR = 2.5
BLK = 2048


def _router_body(x_ref, wt_ref, b_ref, probs_ref, map_ref):
    x = x_ref[...]
    logits = jax.lax.dot_general(
        x, wt_ref[...], (((1,), (1,)), ((), ())),
        preferred_element_type=jnp.float32,
    )
    sp = jnp.maximum(logits, 0.0) + jnp.log(1.0 + jnp.exp(-jnp.abs(logits)))
    scores = jnp.sqrt(sp)
    sel = scores + b_ref[...]
    iota = jax.lax.broadcasted_iota(jnp.int32, sel.shape, 1)
    mask = jnp.zeros(sel.shape, jnp.bool_)
    work = sel
    for _ in range(TOPK):
        idx = jnp.argmax(work, axis=1)[:, None]
        chosen = iota == idx
        mask = jnp.logical_or(mask, chosen)
        work = jnp.where(chosen, -jnp.inf, work)
    w = jnp.where(mask, scores, 0.0)
    denom = jnp.clip(jnp.sum(w, axis=1, keepdims=True), 1e-12, None)
    probs_ref[...] = jnp.where(mask, scores * (TOPK_SCALING_FACTOR / denom), 0.0)
    map_ref[...] = mask


def kernel(hidden, weight, expert_bias):
    flat = hidden.reshape(-1, HIDDEN)
    n = flat.shape[0]
    bias = expert_bias.reshape(1, NUM_EXPERTS)
    probs, rmap = pl.pallas_call(
        _router_body,
        grid=(n // BLK,),
        in_specs=[
            pl.BlockSpec((BLK, HIDDEN), lambda i: (i, 0)),
            pl.BlockSpec((NUM_EXPERTS, HIDDEN), lambda i: (0, 0)),
            pl.BlockSpec((1, NUM_EXPERTS), lambda i: (0, 0)),
        ],
        out_specs=[
            pl.BlockSpec((BLK, NUM_EXPERTS), lambda i: (i, 0)),
            pl.BlockSpec((BLK, NUM_EXPERTS), lambda i: (i, 0)),
        ],
        out_shape=[
            jax.ShapeDtypeStruct((n, NUM_EXPERTS), jnp.float32),
            jax.ShapeDtypeStruct((n, NUM_EXPERTS), jnp.bool_),
        ],
    )(flat, weight, bias)
    return probs, rmap


# speedup vs baseline: 1.7005x; 1.0462x over previous
import jax
import jax.numpy as jnp
from jax.experimental import pallas as pl

HIDDEN = 2048
NUM_EXPERTS = 64
TOPK = 8
TOPK_SCALING_FACTOR = 2.5
BLK = 2048


def _router_body(x_ref, wt_ref, b_ref, probs_ref, map_ref):
    x = x_ref[...]
    logits = jax.lax.dot_general(
        x, wt_ref[...], (((1,), (1,)), ((), ())),
        preferred_element_type=jnp.float32,
    )
    sp = jnp.maximum(logits, 0.0) + jnp.log(1.0 + jnp.exp(-jnp.abs(logits)))
    scores = jnp.sqrt(sp)
    sel = scores + b_ref[...]
    iota = jax.lax.broadcasted_iota(jnp.int32, sel.shape, 1)
    mask = jnp.zeros(sel.shape, jnp.bool_)
    work = sel
    for _ in range(TOPK):
        idx = jnp.argmax(work, axis=1)[:, None]
        chosen = iota == idx
        mask = jnp.logical_or(mask, chosen)
        work = jnp.where(chosen, -jnp.inf, work)
    w = jnp.where(mask, scores, 0.0)
    denom = jnp.clip(jnp.sum(w, axis=1, keepdims=True), 1e-12, None)
    probs_ref[...] = jnp.where(mask, scores * (TOPK_SCALING_FACTOR / denom), 0.0)
    map_ref[...] = mask.astype(jnp.int8)


def kernel(hidden, weight, expert_bias):
    flat = hidden.reshape(-1, HIDDEN)
    n = flat.shape[0]
    bias = expert_bias.reshape(1, NUM_EXPERTS)
    probs, rmap = pl.pallas_call(
        _router_body,
        grid=(n // BLK,),
        in_specs=[
            pl.BlockSpec((BLK, HIDDEN), lambda i: (i, 0)),
            pl.BlockSpec((NUM_EXPERTS, HIDDEN), lambda i: (0, 0)),
            pl.BlockSpec((1, NUM_EXPERTS), lambda i: (0, 0)),
        ],
        out_specs=[
            pl.BlockSpec((BLK, NUM_EXPERTS), lambda i: (i, 0)),
            pl.BlockSpec((BLK, NUM_EXPERTS), lambda i: (i, 0)),
        ],
        out_shape=[
            jax.ShapeDtypeStruct((n, NUM_EXPERTS), jnp.float32),
            jax.ShapeDtypeStruct((n, NUM_EXPERTS), jnp.int8),
        ],
    )(flat, weight, bias)
    return probs, rmap.astype(jnp.bool_)
